# Initial kernel scaffold; baseline (speedup 1.0000x reference)
import jax
import jax.numpy as jnp
from jax.experimental import pallas as pl


def kernel(x):
    raise NotImplementedError("write your pallas kernel here")


# speedup vs baseline: 2.2746x; 2.2746x over previous
import functools

import jax
import jax.numpy as jnp
from jax import lax
from jax.experimental import pallas as pl
from jax.experimental.pallas import tpu as pltpu
from jax.experimental.pallas import tpu_sc as plsc

N_CLASS = 256
LANES = 16
NUM_CORES = 2
NUM_SUBCORES = 16
ZBLK = 65536
SCAT = 128


def _one_hot_sc(x2d, zeros_blk, B, T):
    CT = N_CLASS * T
    n_zero_dmas = CT // ZBLK
    n_groups = T // LANES
    n_scat = T // SCAT

    mesh = plsc.VectorSubcoreMesh(core_axis_name="c", subcore_axis_name="s")

    @functools.partial(
        pl.kernel,
        out_type=jax.ShapeDtypeStruct((B * CT,), jnp.float32),
        mesh=mesh,
        scratch_types=[
            pltpu.VMEM((T,), jnp.int32),
            pltpu.VMEM((ZBLK,), jnp.float32),
            pltpu.VMEM((n_scat, SCAT), jnp.int32),
            pltpu.VMEM((SCAT,), jnp.float32),
            pltpu.SemaphoreType.DMA,
            pltpu.SemaphoreType.DMA,
        ],
    )
    def body(x_hbm, z_hbm, out_hbm, x_v, z_v, idx_v, ones_v, semz, sems):
        b = lax.axis_index("s") * NUM_CORES + lax.axis_index("c")
        base = b * CT
        pltpu.sync_copy(x_hbm.at[b], x_v)
        pltpu.sync_copy(z_hbm, z_v)

        ones16 = jnp.full((LANES,), 1.0, jnp.float32)
        for g in range(SCAT // LANES):
            ones_v[pl.ds(g * LANES, LANES)] = ones16

        zcopies = []
        for j in range(n_zero_dmas):
            cp = pltpu.make_async_copy(
                z_v, out_hbm.at[pl.ds(base + j * ZBLK, ZBLK)], semz
            )
            cp.start()
            zcopies.append(cp)

        iota16 = lax.iota(jnp.int32, LANES)
        per_row = SCAT // LANES
        for g in range(n_groups):
            xv = x_v[pl.ds(g * LANES, LANES)]
            idx = xv * T + (iota16 + (g * LANES)) + base
            idx_v[g // per_row, pl.ds((g % per_row) * LANES, LANES)] = idx

        for cp in zcopies:
            cp.wait()

        scopies = []
        for j in range(n_scat):
            cp = pltpu.make_async_copy(ones_v, out_hbm.at[idx_v.at[j]], sems)
            cp.start()
            scopies.append(cp)
        for cp in scopies:
            cp.wait()

    return body(x2d, zeros_blk).reshape(B, N_CLASS, T)


def kernel(x):
    B = x.shape[0]
    T = x.shape[-1]
    x2d = x.reshape(B, T).astype(jnp.int32)
    zeros_blk = jnp.zeros((ZBLK,), jnp.float32)
    return _one_hot_sc(x2d, zeros_blk, B, T)


# speedup vs baseline: 2.3171x; 1.0187x over previous
import functools

import jax
import jax.numpy as jnp
from jax import lax
from jax.experimental import pallas as pl
from jax.experimental.pallas import tpu as pltpu
from jax.experimental.pallas import tpu_sc as plsc

N_CLASS = 256
LANES = 16
NUM_CORES = 2
NUM_SUBCORES = 16
ZBLK = 65536
SCAT = 128


def _one_hot_sc(x2d, zeros_blk, B, T):
    CT = N_CLASS * T
    n_groups = T // LANES
    n_scat = T // SCAT

    mesh = plsc.VectorSubcoreMesh(core_axis_name="c", subcore_axis_name="s")

    @functools.partial(
        pl.kernel,
        out_type=jax.ShapeDtypeStruct((B * CT,), jnp.float32),
        mesh=mesh,
        scratch_types=[
            pltpu.VMEM((T,), jnp.int32),
            pltpu.VMEM((ZBLK,), jnp.float32),
            pltpu.VMEM_SHARED((CT // 2,), jnp.float32),
            pltpu.VMEM((n_scat, SCAT), jnp.int32),
            pltpu.VMEM((SCAT,), jnp.float32),
            pltpu.SemaphoreType.DMA,
            pltpu.SemaphoreType.DMA,
        ],
    )
    def body(x_hbm, z_hbm, out_hbm, x_v, z_v, zsh, idx_v, ones_v, semz, sems):
        s = lax.axis_index("s")
        b = s * NUM_CORES + lax.axis_index("c")
        base = b * CT
        pltpu.sync_copy(x_hbm.at[b], x_v)
        pltpu.sync_copy(z_hbm, z_v)

        @pl.when(s < 8)
        def _():
            pltpu.sync_copy(z_v, zsh.at[pl.ds(s * ZBLK, ZBLK)])
        plsc.subcore_barrier()

        ones16 = jnp.full((LANES,), 1.0, jnp.float32)
        for g in range(SCAT // LANES):
            ones_v[pl.ds(g * LANES, LANES)] = ones16

        zcp0 = pltpu.make_async_copy(zsh, out_hbm.at[pl.ds(base, CT // 2)], semz)
        zcp1 = pltpu.make_async_copy(zsh, out_hbm.at[pl.ds(base + CT // 2, CT // 2)], semz)
        zcp0.start()
        zcp1.start()

        iota16 = lax.iota(jnp.int32, LANES)
        per_row = SCAT // LANES
        for g in range(n_groups):
            xv = x_v[pl.ds(g * LANES, LANES)]
            idx = xv * T + (iota16 + (g * LANES)) + base
            idx_v[g // per_row, pl.ds((g % per_row) * LANES, LANES)] = idx

        zcp0.wait()
        zcp1.wait()

        scopies = []
        for j in range(n_scat):
            cp = pltpu.make_async_copy(ones_v, out_hbm.at[idx_v.at[j]], sems)
            cp.start()
            scopies.append(cp)
        for cp in scopies:
            cp.wait()

    return body(x2d, zeros_blk).reshape(B, N_CLASS, T)


def kernel(x):
    B = x.shape[0]
    T = x.shape[-1]
    x2d = x.reshape(B, T).astype(jnp.int32)
    zeros_blk = jnp.zeros((ZBLK,), jnp.float32)
    return _one_hot_sc(x2d, zeros_blk, B, T)


# speedup vs baseline: 7.1579x; 3.0891x over previous
import functools

import jax
import jax.numpy as jnp
from jax import lax
from jax.experimental import pallas as pl
from jax.experimental.pallas import tpu as pltpu
from jax.experimental.pallas import tpu_sc as plsc

N_CLASS = 256
LANES = 16
NUM_CORES = 2
NUM_SUBCORES = 16
RCHUNK = 8


def _one_hot_sc(x2d, zeros_blk, B, T):
    n_chunks = N_CLASS // RCHUNK
    n_groups = T // LANES
    cbits = RCHUNK.bit_length() - 1

    mesh = plsc.VectorSubcoreMesh(core_axis_name="c", subcore_axis_name="s")

    @functools.partial(
        pl.kernel,
        out_type=jax.ShapeDtypeStruct((B, N_CLASS, T), jnp.float32),
        mesh=mesh,
        compiler_params=pltpu.CompilerParams(needs_layout_passes=False),
        scratch_types=[
            pltpu.VMEM((T,), jnp.int32),
            pltpu.VMEM((RCHUNK * T,), jnp.float32),
            pltpu.VMEM((RCHUNK * T,), jnp.float32),
            pltpu.SemaphoreType.DMA,
            pltpu.SemaphoreType.DMA,
        ],
    )
    def body(x_hbm, z_hbm, out_hbm, x_v, buf0, buf1, sem0, sem1):
        b = lax.axis_index("s") * NUM_CORES + lax.axis_index("c")
        pltpu.sync_copy(x_hbm.at[b], x_v)
        pltpu.sync_copy(z_hbm, buf0)
        pltpu.sync_copy(z_hbm, buf1)

        ones = jnp.full((LANES,), 1.0, jnp.float32)
        zeros = jnp.zeros((LANES,), jnp.float32)
        iota16 = lax.iota(jnp.int32, LANES)

        bufs = (buf0, buf1)
        sems = (sem0, sem1)
        pending = [None, None]
        for i in range(n_chunks):
            k = i % 2
            buf = bufs[k]
            if pending[k] is not None:
                for cp in pending[k]:
                    cp.wait()

                def scan_unset_set(g):
                    xv = x_v[pl.ds(g * LANES, LANES)]
                    pos = (xv & (RCHUNK - 1)) * T + (iota16 + g * LANES)
                    grp = lax.shift_right_logical(xv, cbits)
                    plsc.store_scatter(buf, [pos], zeros, mask=grp == (i - 2))
                    plsc.store_scatter(buf, [pos], ones, mask=grp == i)

                lax.fori_loop(0, n_groups, lambda g, c: (scan_unset_set(g), c)[1], 0)
            else:

                def scan_set(g):
                    xv = x_v[pl.ds(g * LANES, LANES)]
                    pos = (xv & (RCHUNK - 1)) * T + (iota16 + g * LANES)
                    grp = lax.shift_right_logical(xv, cbits)
                    plsc.store_scatter(buf, [pos], ones, mask=grp == i)

                lax.fori_loop(0, n_groups, lambda g, c: (scan_set(g), c)[1], 0)

            cps = []
            for r in range(RCHUNK):
                cp = pltpu.make_async_copy(
                    buf.at[pl.ds(r * T, T)],
                    out_hbm.at[b, i * RCHUNK + r],
                    sems[k],
                )
                cp.start()
                cps.append(cp)
            pending[k] = cps
        for k in range(2):
            for cp in pending[k]:
                cp.wait()

    return body(x2d, zeros_blk)


def kernel(x):
    B = x.shape[0]
    T = x.shape[-1]
    x2d = x.reshape(B, T).astype(jnp.int32)
    zeros_blk = jnp.zeros((RCHUNK * T,), jnp.float32)
    return _one_hot_sc(x2d, zeros_blk, B, T)


# speedup vs baseline: 7.9206x; 1.1065x over previous
import functools

import jax
import jax.numpy as jnp
from jax import lax
from jax.experimental import pallas as pl
from jax.experimental.pallas import tpu as pltpu
from jax.experimental.pallas import tpu_sc as plsc

N_CLASS = 256
LANES = 16
NUM_CORES = 2
NUM_SUBCORES = 16
RCHUNK = 8


def _one_hot_sc(x3d, zeros_blk, B, T):
    n_chunks = N_CLASS // RCHUNK
    n_groups = T // LANES
    cbits = RCHUNK.bit_length() - 1

    mesh = plsc.VectorSubcoreMesh(core_axis_name="c", subcore_axis_name="s")

    @functools.partial(
        pl.kernel,
        out_type=jax.ShapeDtypeStruct((B, N_CLASS, T), jnp.float32),
        mesh=mesh,
        compiler_params=pltpu.CompilerParams(needs_layout_passes=False),
        scratch_types=[
            pltpu.VMEM((T,), jnp.int32),
            pltpu.VMEM((RCHUNK, T), jnp.float32),
            pltpu.VMEM((RCHUNK, T), jnp.float32),
            pltpu.SemaphoreType.DMA,
            pltpu.SemaphoreType.DMA,
        ],
    )
    def body(x_hbm, z_hbm, out_hbm, x_v, buf0, buf1, sem0, sem1):
        b = lax.axis_index("s") * NUM_CORES + lax.axis_index("c")
        pltpu.sync_copy(x_hbm.at[b, 0], x_v)
        pltpu.sync_copy(z_hbm, buf0)
        pltpu.sync_copy(z_hbm, buf1)

        ones = jnp.full((LANES,), 1.0, jnp.float32)
        zeros = jnp.zeros((LANES,), jnp.float32)
        iota16 = lax.iota(jnp.int32, LANES)

        bufs = (buf0, buf1)
        sems = (sem0, sem1)
        pending = [None, None]
        for i in range(n_chunks):
            k = i % 2
            buf = bufs[k]
            if pending[k] is not None:
                pending[k].wait()

                def scan_unset_set(g):
                    xv = x_v[pl.ds(g * LANES, LANES)]
                    row = xv & (RCHUNK - 1)
                    col = iota16 + g * LANES
                    grp = lax.shift_right_logical(xv, cbits)
                    plsc.store_scatter(buf, [row, col], zeros, mask=grp == (i - 2))
                    plsc.store_scatter(buf, [row, col], ones, mask=grp == i)

                lax.fori_loop(0, n_groups, lambda g, c: (scan_unset_set(g), c)[1], 0)
            else:

                def scan_set(g):
                    xv = x_v[pl.ds(g * LANES, LANES)]
                    row = xv & (RCHUNK - 1)
                    col = iota16 + g * LANES
                    grp = lax.shift_right_logical(xv, cbits)
                    plsc.store_scatter(buf, [row, col], ones, mask=grp == i)

                lax.fori_loop(0, n_groups, lambda g, c: (scan_set(g), c)[1], 0)

            cp = pltpu.make_async_copy(
                buf, out_hbm.at[b, pl.ds(i * RCHUNK, RCHUNK), :], sems[k]
            )
            cp.start()
            pending[k] = cp
        pending[0].wait()
        pending[1].wait()

    return body(x3d, zeros_blk)


def kernel(x):
    B = x.shape[0]
    T = x.shape[-1]
    if x.dtype != jnp.int32:
        x = x.astype(jnp.int32)
    zeros_blk = jnp.zeros((RCHUNK, T), jnp.float32)
    return _one_hot_sc(x, zeros_blk, B, T)


# speedup vs baseline: 8.6286x; 1.0894x over previous
import functools

import jax
import jax.numpy as jnp
from jax import lax
from jax.experimental import pallas as pl
from jax.experimental.pallas import tpu as pltpu
from jax.experimental.pallas import tpu_sc as plsc

N_CLASS = 256
LANES = 16
NUM_CORES = 2
NUM_SUBCORES = 16
RCHUNK = 8


def _one_hot_sc(x3d, B, T):
    n_chunks = N_CLASS // RCHUNK
    n_groups = T // LANES
    cbits = RCHUNK.bit_length() - 1

    mesh = plsc.VectorSubcoreMesh(core_axis_name="c", subcore_axis_name="s")

    @functools.partial(
        pl.kernel,
        out_type=jax.ShapeDtypeStruct((B, N_CLASS, T), jnp.float32),
        mesh=mesh,
        compiler_params=pltpu.CompilerParams(needs_layout_passes=False),
        scratch_types=[
            pltpu.VMEM((T,), jnp.int32),
            pltpu.VMEM((RCHUNK, T), jnp.float32),
            pltpu.VMEM((RCHUNK, T), jnp.float32),
            pltpu.SemaphoreType.DMA,
            pltpu.SemaphoreType.DMA,
            pltpu.SemaphoreType.DMA,
        ],
    )
    def body(x_hbm, out_hbm, x_v, buf0, buf1, sem0, sem1, semx):
        b = lax.axis_index("s") * NUM_CORES + lax.axis_index("c")
        xcp = pltpu.make_async_copy(x_hbm.at[b, 0], x_v, semx)
        xcp.start()

        ones = jnp.full((LANES,), 1.0, jnp.float32)
        zeros = jnp.zeros((LANES,), jnp.float32)
        iota16 = lax.iota(jnp.int32, LANES)
        bufs = (buf0, buf1)
        sems = (sem0, sem1)

        def zero_cols(g, c):
            for r in range(RCHUNK):
                buf0[r, pl.ds(g * LANES, LANES)] = zeros
                buf1[r, pl.ds(g * LANES, LANES)] = zeros
            return c

        lax.fori_loop(0, n_groups, zero_cols, 0)
        xcp.wait()

        def scan(buf, i, i_unset):
            def group(g, c):
                xv = x_v[pl.ds(g * LANES, LANES)]
                row = xv & (RCHUNK - 1)
                col = iota16 + g * LANES
                grp = lax.shift_right_logical(xv, cbits)
                if i_unset is not None:
                    plsc.store_scatter(buf, [row, col], zeros, mask=grp == i_unset)
                plsc.store_scatter(buf, [row, col], ones, mask=grp == i)
                return c

            lax.fori_loop(0, n_groups, group, 0)

        def start_out(buf, i, sem):
            cp = pltpu.make_async_copy(
                buf, out_hbm.at[b, pl.ds(i * RCHUNK, RCHUNK), :], sem
            )
            cp.start()
            return cp

        scan(buf0, 0, None)
        start_out(buf0, 0, sem0)
        scan(buf1, 1, None)
        start_out(buf1, 1, sem1)

        def pair(p, c):
            for k in range(2):
                i = 2 * p + k
                buf, sem = bufs[k], sems[k]
                pltpu.make_async_copy(
                    buf, out_hbm.at[b, pl.ds((i - 2) * RCHUNK, RCHUNK), :], sem
                ).wait()
                scan(buf, i, i - 2)
                start_out(buf, i, sem)
            return c

        lax.fori_loop(1, n_chunks // 2, pair, 0)

        for k in range(2):
            i = n_chunks - 2 + k
            pltpu.make_async_copy(
                bufs[k], out_hbm.at[b, pl.ds(i * RCHUNK, RCHUNK), :], sems[k]
            ).wait()

    return body(x3d)


def kernel(x):
    B = x.shape[0]
    T = x.shape[-1]
    if x.dtype != jnp.int32:
        x = x.astype(jnp.int32)
    return _one_hot_sc(x, B, T)
